# baseline (speedup 1.0000x reference)
import jax
import jax.numpy as jnp
from jax import lax
from jax.experimental import pallas as pl
from jax.experimental.pallas import tpu as pltpu

_BLOCK_COLS = 2048

_DN = (((1,), (0,)), ((), ()))


def _wx(w, x):
    return lax.dot_general(w.astype(jnp.bfloat16), x.astype(jnp.bfloat16),
                           _DN, preferred_element_type=jnp.float32)


def _fused_mlp_body(xt_ref, w0a_ref, b0a_ref, w0b_ref, b0b_ref, w1pre_ref,
                    b1pre_ref, w1post_ref, b1post_ref, out_ref):
    xa = xt_ref[0:128, :]
    xb = xt_ref[128:256, :]
    m0 = xt_ref[256:257, :] != 0.0
    m1 = xt_ref[257:258, :] != 0.0

    b0a = b0a_ref[...].T
    b0b = b0b_ref[...].T
    b1pre = b1pre_ref[...].T
    b1post = b1post_ref[...].T

    h = jnp.maximum(_wx(w0a_ref[...], xa) + b0a, 0.0)
    x0 = jnp.maximum(_wx(w0b_ref[...], h) + b0b, 0.0)
    x0 = jnp.where(m0, x0, 0.0)

    h1 = jnp.maximum(_wx(w1pre_ref[...], xb) + b1pre, 0.0)

    out = (_wx(w1post_ref[:, 0:128], x0) + _wx(w1post_ref[:, 128:384], h1)
           + b1post)
    out = jnp.where(m1, out, 0.0)
    out_ref[...] = out.T


@jax.jit
def kernel(input_val, W0a, b0a, W0b, b0b, W1pre, b1pre, W1post, b1post):
    n, d_in = input_val.shape
    block = min(_BLOCK_COLS, n)
    grid = (n // block,)

    xt = input_val.T
    full = lambda w: pl.BlockSpec(w.shape, lambda j: (0,) * w.ndim)
    b0a2 = b0a.reshape(1, 256)
    b0b2 = b0b.reshape(1, 128)
    b1pre2 = b1pre.reshape(1, 256)
    b1post2 = b1post.reshape(1, 128)
    out = pl.pallas_call(
        _fused_mlp_body,
        grid=grid,
        in_specs=[
            pl.BlockSpec((d_in, block), lambda j: (0, j)),
            full(W0a), full(b0a2), full(W0b), full(b0b2),
            full(W1pre), full(b1pre2), full(W1post), full(b1post2),
        ],
        out_specs=pl.BlockSpec((block, 128), lambda j: (j, 0)),
        out_shape=jax.ShapeDtypeStruct((n, 128), input_val.dtype),
        compiler_params=pltpu.CompilerParams(
            dimension_semantics=("parallel",),
            vmem_limit_bytes=100 * 1024 * 1024,
        ),
    )(xt, W0a, b0a2, W0b, b0b2, W1pre, b1pre2, W1post, b1post2)
    return out


# speedup vs baseline: 1.1580x; 1.1580x over previous
import jax
import jax.numpy as jnp
from jax import lax
from jax.experimental import pallas as pl
from jax.experimental.pallas import tpu as pltpu

_BLOCK_COLS = 4096

_DN = (((1,), (0,)), ((), ()))


def _wx(w, x):
    return lax.dot_general(w.astype(jnp.bfloat16), x.astype(jnp.bfloat16),
                           _DN, preferred_element_type=jnp.float32)


def _fused_mlp_body(xt_ref, w0a_ref, b0a_ref, w0b_ref, b0b_ref, w1pre_ref,
                    b1pre_ref, w1post_ref, b1post_ref, out_ref):
    xa = xt_ref[0:128, :]
    xb = xt_ref[128:256, :]
    m0 = xt_ref[256:257, :] != 0.0
    m1 = xt_ref[257:258, :] != 0.0

    b0a = b0a_ref[...].T
    b0b = b0b_ref[...].T
    b1pre = b1pre_ref[...].T
    b1post = b1post_ref[...].T

    h = jnp.maximum(_wx(w0a_ref[...], xa) + b0a, 0.0)
    x0 = jnp.maximum(_wx(w0b_ref[...], h) + b0b, 0.0)
    x0 = jnp.where(m0, x0, 0.0)

    h1 = jnp.maximum(_wx(w1pre_ref[...], xb) + b1pre, 0.0)

    out = (_wx(w1post_ref[:, 0:128], x0) + _wx(w1post_ref[:, 128:384], h1)
           + b1post)
    out = jnp.where(m1, out, 0.0)
    out_ref[...] = out.T


@jax.jit
def kernel(input_val, W0a, b0a, W0b, b0b, W1pre, b1pre, W1post, b1post):
    n, d_in = input_val.shape
    block = min(_BLOCK_COLS, n)
    grid = (n // block,)

    xt = input_val.T
    full = lambda w: pl.BlockSpec(w.shape, lambda j: (0,) * w.ndim)
    b0a2 = b0a.reshape(1, 256)
    b0b2 = b0b.reshape(1, 128)
    b1pre2 = b1pre.reshape(1, 256)
    b1post2 = b1post.reshape(1, 128)
    out = pl.pallas_call(
        _fused_mlp_body,
        grid=grid,
        in_specs=[
            pl.BlockSpec((d_in, block), lambda j: (0, j)),
            full(W0a), full(b0a2), full(W0b), full(b0b2),
            full(W1pre), full(b1pre2), full(W1post), full(b1post2),
        ],
        out_specs=pl.BlockSpec((block, 128), lambda j: (j, 0)),
        out_shape=jax.ShapeDtypeStruct((n, 128), input_val.dtype),
        compiler_params=pltpu.CompilerParams(
            dimension_semantics=("parallel",),
            vmem_limit_bytes=100 * 1024 * 1024,
        ),
    )(xt, W0a, b0a2, W0b, b0b2, W1pre, b1pre2, W1post, b1post2)
    return out
